# baseline (speedup 1.0000x reference)
import jax
import jax.numpy as jnp
from jax.experimental import pallas as pl
from jax.experimental.pallas import tpu as pltpu

_N_CLUSTERS = 16384
_TM = 1024
_TN = 1024
_BIG = 2**30


def _assign_kernel(x_ref, c_ref, mask_ref, out_ref, bestv_ref, besti_ref):
    j = pl.program_id(1)
    nj = pl.num_programs(1)

    x = x_ref[...]
    c = c_ref[...]
    s = jax.lax.dot_general(
        x, c, (((1,), (1,)), ((), ())),
        preferred_element_type=jnp.float32,
    )
    x_sq = jnp.sum(x * x, axis=1, keepdims=True)
    c_sq = jnp.sum(c * c, axis=1)
    dist = (x_sq - 2.0 * s) + c_sq[None, :]

    local_min = jnp.min(dist, axis=1)
    idx = jax.lax.broadcasted_iota(jnp.int32, dist.shape, 1)
    local_arg = jnp.min(
        jnp.where(dist == local_min[:, None], idx, _BIG), axis=1
    ) + j * _TN

    @pl.when(j == 0)
    def _init():
        bestv_ref[...] = local_min
        besti_ref[...] = local_arg

    @pl.when(j > 0)
    def _update():
        bv = bestv_ref[...]
        upd = local_min < bv
        bestv_ref[...] = jnp.where(upd, local_min, bv)
        besti_ref[...] = jnp.where(upd, local_arg, besti_ref[...])

    @pl.when(j == nj - 1)
    def _write():
        out_ref[...] = jnp.where(mask_ref[...] != 0, besti_ref[...],
                                 jnp.int32(-1))


def kernel(activations, attention_mask, centroids):
    orig_shape = attention_mask.shape
    d = activations.shape[-1]
    x = activations.reshape(-1, d)
    n_tokens = x.shape[0]
    mask = attention_mask.reshape(-1).astype(jnp.int32)

    grid = (n_tokens // _TM, _N_CLUSTERS // _TN)
    labels = pl.pallas_call(
        _assign_kernel,
        grid=grid,
        in_specs=[
            pl.BlockSpec((_TM, d), lambda i, j: (i, 0)),
            pl.BlockSpec((_TN, d), lambda i, j: (j, 0)),
            pl.BlockSpec((_TM,), lambda i, j: (i,)),
        ],
        out_specs=pl.BlockSpec((_TM,), lambda i, j: (i,)),
        out_shape=jax.ShapeDtypeStruct((n_tokens,), jnp.int32),
        scratch_shapes=[
            pltpu.VMEM((_TM,), jnp.float32),
            pltpu.VMEM((_TM,), jnp.int32),
        ],
        compiler_params=pltpu.CompilerParams(
            dimension_semantics=("parallel", "arbitrary"),
        ),
    )(x, centroids, mask)
    return labels.reshape(orig_shape)


# speedup vs baseline: 1.0117x; 1.0117x over previous
import jax
import jax.numpy as jnp
from jax.experimental import pallas as pl
from jax.experimental.pallas import tpu as pltpu

_N_CLUSTERS = 16384
_TM = 2048
_TN = 1024
_BIG = 2**30


def _assign_kernel(x_ref, c_ref, mask_ref, out_ref, bestv_ref, besti_ref):
    j = pl.program_id(1)
    nj = pl.num_programs(1)

    x = x_ref[...]
    c = c_ref[...]
    s = jax.lax.dot_general(
        x, c, (((1,), (1,)), ((), ())),
        preferred_element_type=jnp.float32,
    )
    x_sq = jnp.sum(x * x, axis=1, keepdims=True)
    c_sq = jnp.sum(c * c, axis=1)
    dist = (x_sq - 2.0 * s) + c_sq[None, :]

    local_min = jnp.min(dist, axis=1)
    idx = jax.lax.broadcasted_iota(jnp.int32, dist.shape, 1)
    local_arg = jnp.min(
        jnp.where(dist == local_min[:, None], idx, _BIG), axis=1
    ) + j * _TN

    @pl.when(j == 0)
    def _init():
        bestv_ref[...] = local_min
        besti_ref[...] = local_arg

    @pl.when(j > 0)
    def _update():
        bv = bestv_ref[...]
        upd = local_min < bv
        bestv_ref[...] = jnp.where(upd, local_min, bv)
        besti_ref[...] = jnp.where(upd, local_arg, besti_ref[...])

    @pl.when(j == nj - 1)
    def _write():
        out_ref[...] = jnp.where(mask_ref[...] != 0, besti_ref[...],
                                 jnp.int32(-1))


def kernel(activations, attention_mask, centroids):
    orig_shape = attention_mask.shape
    d = activations.shape[-1]
    x = activations.reshape(-1, d)
    n_tokens = x.shape[0]
    mask = attention_mask.reshape(-1).astype(jnp.int32)

    grid = (n_tokens // _TM, _N_CLUSTERS // _TN)
    labels = pl.pallas_call(
        _assign_kernel,
        grid=grid,
        in_specs=[
            pl.BlockSpec((_TM, d), lambda i, j: (i, 0)),
            pl.BlockSpec((_TN, d), lambda i, j: (j, 0)),
            pl.BlockSpec((_TM,), lambda i, j: (i,)),
        ],
        out_specs=pl.BlockSpec((_TM,), lambda i, j: (i,)),
        out_shape=jax.ShapeDtypeStruct((n_tokens,), jnp.int32),
        scratch_shapes=[
            pltpu.VMEM((_TM,), jnp.float32),
            pltpu.VMEM((_TM,), jnp.int32),
        ],
        compiler_params=pltpu.CompilerParams(
            dimension_semantics=("parallel", "arbitrary"),
        ),
    )(x, centroids, mask)
    return labels.reshape(orig_shape)


# speedup vs baseline: 1.1277x; 1.1147x over previous
import jax
import jax.numpy as jnp
from jax.experimental import pallas as pl
from jax.experimental.pallas import tpu as pltpu

_N_CLUSTERS = 16384
_TM = 2048
_TN = 1024
_BIG = 2**30


def _assign_kernel(x_ref, c_ref, mask_ref, out_ref, bestv_ref, besti_ref):
    j = pl.program_id(1)
    nj = pl.num_programs(1)

    x = x_ref[...]
    c = c_ref[...]
    c_sq = jnp.sum(c * c, axis=1)
    s = jax.lax.dot_general(
        x, c * -2.0, (((1,), (1,)), ((), ())),
        preferred_element_type=jnp.float32,
    )
    dist = s + c_sq[None, :]

    local_min = jnp.min(dist, axis=1)
    idx = jax.lax.broadcasted_iota(jnp.int32, dist.shape, 1)
    local_arg = jnp.min(
        jnp.where(dist == local_min[:, None], idx, _BIG), axis=1
    ) + j * _TN

    @pl.when(j == 0)
    def _init():
        bestv_ref[...] = local_min
        besti_ref[...] = local_arg

    @pl.when(j > 0)
    def _update():
        bv = bestv_ref[...]
        upd = local_min < bv
        bestv_ref[...] = jnp.where(upd, local_min, bv)
        besti_ref[...] = jnp.where(upd, local_arg, besti_ref[...])

    @pl.when(j == nj - 1)
    def _write():
        out_ref[...] = jnp.where(mask_ref[...] != 0, besti_ref[...],
                                 jnp.int32(-1))


def kernel(activations, attention_mask, centroids):
    orig_shape = attention_mask.shape
    d = activations.shape[-1]
    x = activations.reshape(-1, d)
    n_tokens = x.shape[0]
    mask = attention_mask.reshape(-1).astype(jnp.int32)

    grid = (n_tokens // _TM, _N_CLUSTERS // _TN)
    labels = pl.pallas_call(
        _assign_kernel,
        grid=grid,
        in_specs=[
            pl.BlockSpec((_TM, d), lambda i, j: (i, 0)),
            pl.BlockSpec((_TN, d), lambda i, j: (j, 0)),
            pl.BlockSpec((_TM,), lambda i, j: (i,)),
        ],
        out_specs=pl.BlockSpec((_TM,), lambda i, j: (i,)),
        out_shape=jax.ShapeDtypeStruct((n_tokens,), jnp.int32),
        scratch_shapes=[
            pltpu.VMEM((_TM,), jnp.float32),
            pltpu.VMEM((_TM,), jnp.int32),
        ],
        compiler_params=pltpu.CompilerParams(
            dimension_semantics=("parallel", "arbitrary"),
        ),
    )(x, centroids, mask)
    return labels.reshape(orig_shape)
